# Initial kernel scaffold; baseline (speedup 1.0000x reference)
import jax
import jax.numpy as jnp
from jax.experimental import pallas as pl


def kernel(input, embed, rw1_0, rb1_0, rw2_0, rb2_0, rw1_1, rb1_1, rw2_1, rb2_1, rw1_2, rb1_2, rw2_2, rb2_2, wf, bf):
    raise NotImplementedError("write your pallas kernel here")


# speedup vs baseline: 1.2033x; 1.2033x over previous
import functools

import jax
import jax.numpy as jnp
from jax import lax
from jax.experimental import pallas as pl
from jax.experimental.pallas import tpu as pltpu
from jax.experimental.pallas import tpu_sc as plsc

_B, _C, _H, _W = 8, 64, 32, 32
_NTOK = 8192
_N = _B * _H * _W
_TBLK = 1024
_KBLK = 2048
_PAD = 64


def _argmin_body(f_ref, e_ref, idx_ref, bv_ref, bi_ref):
    k = pl.program_id(1)
    nk = pl.num_programs(1)
    f = f_ref[...]
    e = e_ref[...]
    af = (2.0 * f).astype(jnp.bfloat16)
    eb = e.astype(jnp.bfloat16)
    m = lax.dot_general(af, eb, (((1,), (0,)), ((), ())),
                        preferred_element_type=jnp.float32)
    f2 = jnp.sum(f * f, axis=1, keepdims=True)
    e2 = jnp.sum(e * e, axis=0, keepdims=True)
    dist = (f2 - m) + e2
    v = jnp.min(dist, axis=1, keepdims=True)
    vr = v.astype(jnp.bfloat16).astype(jnp.float32)
    jg = lax.broadcasted_iota(jnp.int32, (_TBLK, _KBLK), 1)
    li = jnp.min(jnp.where(dist == v, jg, jnp.int32(2 ** 30)),
                 axis=1, keepdims=True) + k * _KBLK

    @pl.when(k == 0)
    def _():
        bv_ref[...] = vr
        bi_ref[...] = li

    @pl.when(k > 0)
    def _():
        better = v < bv_ref[...]
        bv_ref[...] = jnp.where(better, vr, bv_ref[...])
        bi_ref[...] = jnp.where(better, li, bi_ref[...])

    @pl.when(k == nk - 1)
    def _():
        idx_ref[...] = bi_ref[...]


def _vq_argmin(zf, embed):
    return pl.pallas_call(
        _argmin_body,
        grid=(_N // _TBLK, _NTOK // _KBLK),
        in_specs=[
            pl.BlockSpec((_TBLK, _C), lambda t, k: (t, 0)),
            pl.BlockSpec((_C, _KBLK), lambda t, k: (0, k)),
        ],
        out_specs=pl.BlockSpec((_TBLK, 1), lambda t, k: (t, 0)),
        out_shape=jax.ShapeDtypeStruct((_N, 1), jnp.int32),
        scratch_shapes=[
            pltpu.VMEM((_TBLK, 1), jnp.float32),
            pltpu.VMEM((_TBLK, 1), jnp.int32),
        ],
        compiler_params=pltpu.CompilerParams(
            dimension_semantics=("parallel", "arbitrary")),
    )(zf, embed)


def _sc_body(table_hbm, idx_hbm, q_hbm, wf_hbm,
             idx_v, rows_v, vals_v, idxadj_v, stage_v, hist_sh, sem):
    c = lax.axis_index("c")
    s = lax.axis_index("s")
    wid = c * 16 + s
    base = wid * 256

    for t in range(2):
        pltpu.sync_copy(idx_hbm.at[pl.ds(base + t * 128, 128)], idx_v.at[t])
    for t in range(2):
        pltpu.async_copy(table_hbm.at[idx_v.at[t]],
                         rows_v.at[pl.ds(t * 128, 128)], sem).wait()
    pltpu.sync_copy(rows_v, q_hbm.at[pl.ds(base, 256)])

    off = (s // 4) * _NTOK
    inv = jnp.full((16,), 1.0 / 1024.0, jnp.float32)
    for t in range(2):
        for i in range(8):
            sl = pl.ds(i * 16, 16)
            idxadj_v[t, sl] = idx_v[t, sl] + off
            vals_v[t, sl] = inv

    zero = jnp.zeros((16,), jnp.float32)
    for i in range(128):
        stage_v[pl.ds(i * 16, 16)] = zero
    pltpu.sync_copy(stage_v, hist_sh.at[pl.ds(s * 2048, 2048)])
    plsc.subcore_barrier()
    for t in range(2):
        pltpu.sync_copy(vals_v.at[t], hist_sh.at[idxadj_v.at[t]], add=True)
    plsc.subcore_barrier()
    pltpu.sync_copy(hist_sh.at[pl.ds(s * 2048, 2048)], stage_v)
    pltpu.sync_copy(stage_v, wf_hbm.at[pl.ds(c * 32768 + s * 2048, 2048)])


def _sc_gather_hist(table, idx):
    mesh = plsc.VectorSubcoreMesh(core_axis_name="c", subcore_axis_name="s")
    run = pl.kernel(
        _sc_body,
        out_type=(jax.ShapeDtypeStruct((_N, 128), jnp.float32),
                  jax.ShapeDtypeStruct((_B * _NTOK,), jnp.float32)),
        mesh=mesh,
        scratch_types=[
            pltpu.VMEM((2, 128), jnp.int32),
            pltpu.VMEM((256, 128), jnp.float32),
            pltpu.VMEM((2, 128), jnp.float32),
            pltpu.VMEM((2, 128), jnp.int32),
            pltpu.VMEM((2048,), jnp.float32),
            pltpu.VMEM_SHARED((4 * _NTOK,), jnp.float32),
            pltpu.SemaphoreType.DMA,
        ],
    )
    return run(table, idx)


def _conv_body(q_ref, z_ref, w1_ref, b1_ref, w2_ref, b2_ref, wf_ref, bf_ref,
               quant_ref, diff_ref, xp_ref, h_ref):
    r = lax.broadcasted_iota(jnp.int32, (_N, 1), 0)
    ii = (r // 32) % 32
    jj = r % 32
    h_ref[...] = q_ref[...]
    zeros_pad = jnp.zeros((_PAD, _C), jnp.float32)
    for blk in range(3):
        xp_ref[pl.ds(0, _PAD), :] = zeros_pad
        xp_ref[pl.ds(_PAD + _N, _PAD), :] = zeros_pad
        xp_ref[pl.ds(_PAD, _N), :] = jnp.maximum(h_ref[...], 0.0)
        acc = jnp.zeros((_N, 16), jnp.float32)
        for di in range(3):
            for dj in range(3):
                shift = (di - 1) * 32 + (dj - 1)
                xs = xp_ref[pl.ds(_PAD + shift, _N), :]
                iv = ii + (di - 1)
                jv = jj + (dj - 1)
                msk = (iv >= 0) & (iv < 32) & (jv >= 0) & (jv < 32)
                xm = jnp.where(msk, xs, 0.0)
                wblk = w1_ref[pl.ds((blk * 9 + di * 3 + dj) * _C, _C), :]
                acc = acc + lax.dot_general(
                    xm, wblk, (((1,), (0,)), ((), ())),
                    preferred_element_type=jnp.float32)
        y = jnp.maximum(acc + b1_ref[pl.ds(blk, 1), :], 0.0)
        out = lax.dot_general(
            y, w2_ref[pl.ds(blk * 16, 16), :], (((1,), (0,)), ((), ())),
            preferred_element_type=jnp.float32) + b2_ref[pl.ds(blk, 1), :]
        h_ref[...] = h_ref[...] + out
    quant = jnp.sum(h_ref[...] * wf_ref[...], axis=1, keepdims=True) \
        + bf_ref[...]
    quant_ref[...] = quant
    d = q_ref[...] - z_ref[...]
    diff_ref[...] = jnp.sum(d * d, axis=(0, 1), keepdims=True) \
        * jnp.float32(1.0 / (_N * _C))


def _conv_stack(q, zf, w1, b1, w2, b2, wfr, bfr):
    full = lambda shape: pl.BlockSpec(shape, lambda: tuple(0 for _ in shape))
    return pl.pallas_call(
        _conv_body,
        in_specs=[
            full((_N, _C)), full((_N, _C)),
            full((27 * _C, 16)), full((3, 16)),
            full((3 * 16, _C)), full((3, _C)),
            full((1, _C)), full((1, 1)),
        ],
        out_specs=[full((_N, 1)), full((1, 1))],
        out_shape=(jax.ShapeDtypeStruct((_N, 1), jnp.float32),
                   jax.ShapeDtypeStruct((1, 1), jnp.float32)),
        scratch_shapes=[
            pltpu.VMEM((_N + 2 * _PAD, _C), jnp.float32),
            pltpu.VMEM((_N, _C), jnp.float32),
        ],
    )(q, zf, w1, b1, w2, b2, wfr, bfr)


def kernel(input, embed, rw1_0, rb1_0, rw2_0, rb2_0, rw1_1, rb1_1, rw2_1,
           rb2_1, rw1_2, rb1_2, rw2_2, rb2_2, wf, bf):
    zf = jnp.transpose(input, (0, 2, 3, 1)).reshape(_N, _C)
    idx = _vq_argmin(zf, embed).reshape(_N)
    table = jnp.zeros((_NTOK, 128), jnp.float32).at[:, :_C].set(embed.T)
    q128, wf_flat = _sc_gather_hist(table, idx)
    q = q128[:, :_C]

    w1 = jnp.concatenate(
        [w.transpose(2, 3, 1, 0).reshape(9 * _C, 16)
         for w in (rw1_0, rw1_1, rw1_2)], axis=0)
    b1 = jnp.stack([rb1_0, rb1_1, rb1_2])
    w2 = jnp.concatenate(
        [w[:, :, 0, 0].T for w in (rw2_0, rw2_1, rw2_2)], axis=0)
    b2 = jnp.stack([rb2_0, rb2_1, rb2_2])
    wfr = wf[:, :, 0, 0]
    bfr = bf.reshape(1, 1)

    quant_flat, diff2 = _conv_stack(q, zf, w1, b1, w2, b2, wfr, bfr)
    quant = quant_flat.reshape(_B, 1, _H, _W)
    diff = diff2.reshape(1)
    wordfreq = wf_flat.reshape(_B, _NTOK)
    return (quant, diff, wordfreq)
